# Initial kernel scaffold; baseline (speedup 1.0000x reference)
import jax
import jax.numpy as jnp
from jax.experimental import pallas as pl


def kernel(neuron_ids, embedding_weight):
    raise NotImplementedError("write your pallas kernel here")


# speedup vs baseline: 9.1749x; 9.1749x over previous
import functools

import jax
import jax.numpy as jnp
from jax import lax
from jax.experimental import pallas as pl
from jax.experimental.pallas import tpu as pltpu
from jax.experimental.pallas import tpu_sc as plsc

_EMB = 128
_CHUNK = 128
_NB = 4
_NC = 2
_NS = 16
_NW = _NC * _NS


@functools.lru_cache(maxsize=None)
def _build(B: int):
    per_w = B // _NW
    G = per_w // _CHUNK
    R = G // _NB

    mesh = plsc.VectorSubcoreMesh(
        core_axis_name="c", subcore_axis_name="s",
        num_cores=_NC, num_subcores=_NS,
    )

    @functools.partial(
        pl.kernel,
        out_type=jax.ShapeDtypeStruct((B, _EMB), jnp.float32),
        mesh=mesh,
        scratch_types=[
            pltpu.VMEM((G, _CHUNK), jnp.int32),
            pltpu.VMEM((_NB, _CHUNK, _EMB), jnp.float32),
        ] + [pltpu.SemaphoreType.DMA] * (1 + 2 * _NB),
    )
    def gather_kernel(idx_hbm, table_hbm, out_hbm, idx_v, rows_v, *sems):
        isem = sems[0]
        gsems = sems[1:1 + _NB]
        osems = sems[1 + _NB:]

        wid = lax.axis_index("s") * _NC + lax.axis_index("c")
        base = wid * per_w

        pltpu.async_copy(idx_hbm.at[wid], idx_v, isem).wait()

        def gather(g, b):
            return pltpu.make_async_copy(
                table_hbm.at[idx_v.at[g]], rows_v.at[b], gsems[b])

        def out(g, b):
            return pltpu.make_async_copy(
                rows_v.at[b],
                out_hbm.at[pl.ds(base + g * _CHUNK, _CHUNK)],
                osems[b])

        for b in range(_NB):
            gather(b, b).start()

        def round_body(r, carry):
            g0 = r * _NB
            for b in range(_NB):
                gather(g0 + b, b).wait()
                out(g0 + b, b).start()

            @pl.when(r < R - 1)
            def _():
                for b in range(_NB):
                    out(g0 + b, b).wait()
                    gather(g0 + _NB + b, b).start()

            @pl.when(r == R - 1)
            def _():
                for b in range(_NB):
                    out(g0 + b, b).wait()

            return carry

        lax.fori_loop(0, R, round_body, 0)

    return gather_kernel


def kernel(neuron_ids, embedding_weight):
    batch, hist = neuron_ids.shape
    B = batch * hist
    per_w = B // _NW
    G = per_w // _CHUNK
    idx = neuron_ids.astype(jnp.int32).reshape(_NW, G, _CHUNK)
    out = _build(B)(idx, embedding_weight)
    return out.reshape(batch, hist, _EMB)
